# baseline (speedup 1.0000x reference)
import jax
import jax.numpy as jnp
from jax.experimental import pallas as pl
from jax.experimental.pallas import tpu as pltpu

TAU = 2.0
VTH = 1.0
EPS = 1e-5
DIM = 64
NUM_HEADS = 4
N_WIN = 8
TOPK = 4


def _lif(v):
    return jnp.where(v / TAU - VTH >= 0.0, 1.0, 0.0).astype(v.dtype)


def _bn_cols(x, g, b):
    m = jnp.mean(x, axis=0, keepdims=True)
    v = jnp.mean(jnp.square(x - m), axis=0, keepdims=True)
    return (x - m) / jnp.sqrt(v + EPS) * g + b


def _tf_stack_kernel(tb, xin_ref, *args):
    wrefs = args[:30]
    out_ref = args[30]
    qkv_s = args[31]
    att_s = args[32]

    C = DIM
    H = NUM_HEADS
    dh = C // H
    N = 16
    w = N // N_WIN

    x = _lif(xin_ref[...])

    r8 = jax.lax.broadcasted_iota(jnp.int32, (8, 16), 0)
    c16 = jax.lax.broadcasted_iota(jnp.int32, (8, 16), 1)
    P = jnp.where(c16 // w == r8, 1.0 / w, 0.0)
    r16 = jax.lax.broadcasted_iota(jnp.int32, (16, 8), 0)
    c8 = jax.lax.broadcasted_iota(jnp.int32, (16, 8), 1)
    E = jnp.where(r16 // w == c8, 1.0, 0.0)
    colj = jax.lax.broadcasted_iota(jnp.int32, (8, 8), 1)

    for blk in range(3):
        g1, b1, Wqkv, Wo, g2, b2, W1, bb1, W2, bb2 = wrefs[blk * 10:(blk + 1) * 10]
        y = _bn_cols(x, g1[...], b1[...])
        qkv_s[...] = jnp.dot(y, Wqkv[...], preferred_element_type=jnp.float32)

        def body(t, carry):
            base = t * N
            qt = qkv_s[pl.ds(base, N), 0:C]
            kt = qkv_s[pl.ds(base, N), C:2 * C]
            vt = qkv_s[pl.ds(base, N), 2 * C:3 * C]
            qp = jnp.dot(P, qt, preferred_element_type=jnp.float32)
            kp = jnp.dot(P, kt, preferred_element_type=jnp.float32)
            r = jax.lax.dot_general(qp, kp, (((1,), (1,)), ((), ())),
                                    preferred_element_type=jnp.float32)
            cnt = jnp.zeros((8, 8), jnp.float32)
            for kk in range(8):
                rk = r[:, kk:kk + 1]
                gt = jnp.where(rk > r, 1.0, 0.0)
                eq = jnp.where((rk == r) & (colj > kk), 1.0, 0.0)
                cnt = cnt + gt + eq
            keep = jnp.where(cnt < float(TOPK), 1.0, 0.0)
            mtok = jax.lax.dot_general(
                jnp.dot(E, keep, preferred_element_type=jnp.float32), E,
                (((1,), (1,)), ((), ())), preferred_element_type=jnp.float32)
            amask = (mtok - 1.0) * 1e30
            for h in range(H):
                qh = qt[:, h * dh:(h + 1) * dh]
                kh = kt[:, h * dh:(h + 1) * dh]
                vh = vt[:, h * dh:(h + 1) * dh]
                s = jax.lax.dot_general(qh, kh, (((1,), (1,)), ((), ())),
                                        preferred_element_type=jnp.float32)
                s = s / 4.0 + amask
                s = s - jnp.max(s, axis=1, keepdims=True)
                e = jnp.exp(s)
                p = e / jnp.sum(e, axis=1, keepdims=True)
                att_s[pl.ds(base, N), h * dh:(h + 1) * dh] = jnp.dot(
                    p, vh, preferred_element_type=jnp.float32)
            return carry

        jax.lax.fori_loop(0, tb, body, 0)
        x = x + jnp.dot(att_s[...], Wo[...], preferred_element_type=jnp.float32)
        z = _bn_cols(x, g2[...], b2[...])
        f = _lif(jnp.dot(z, W1[...], preferred_element_type=jnp.float32) + bb1[...])
        x = x + _lif(jnp.dot(f, W2[...], preferred_element_type=jnp.float32) + bb2[...])

    out_ref[...] = x


def _tf_stack(xsnn_pre, blocks):
    rows = xsnn_pre.shape[0]
    tb = rows // 16
    wargs = []
    for blk in blocks:
        wargs += [blk['bn1_g'].reshape(1, DIM), blk['bn1_b'].reshape(1, DIM),
                  blk['Wqkv'], blk['Wo'],
                  blk['bn2_g'].reshape(1, DIM), blk['bn2_b'].reshape(1, DIM),
                  blk['ffn1_w'], blk['ffn1_b'].reshape(1, 4 * DIM),
                  blk['ffn2_w'], blk['ffn2_b'].reshape(1, DIM)]
    import functools
    return pl.pallas_call(
        functools.partial(_tf_stack_kernel, tb),
        out_shape=jax.ShapeDtypeStruct((rows, DIM), jnp.float32),
        scratch_shapes=[pltpu.VMEM((rows, 3 * DIM), jnp.float32),
                        pltpu.VMEM((rows, DIM), jnp.float32)],
    )(xsnn_pre, *wargs)


def _conv3d(x, w, b, stride, padding):
    y = jax.lax.conv_general_dilated(x, w, window_strides=stride, padding=padding,
                                     dimension_numbers=('NCDHW', 'OIDHW', 'NCDHW'))
    return y + b[None, :, None, None, None]


def _bn3d(x, g, b):
    m = x.mean(axis=(0, 2, 3, 4), keepdims=True)
    v = x.var(axis=(0, 2, 3, 4), keepdims=True)
    return (x - m) / jnp.sqrt(v + EPS) * g[None, :, None, None, None] + b[None, :, None, None, None]


def _maxpool_122(x):
    return jax.lax.reduce_window(x, -jnp.inf, jax.lax.max, (1, 1, 1, 2, 2), (1, 1, 1, 2, 2), 'VALID')


def kernel(x, params):
    p = params
    b = x.shape[0]
    x = _conv3d(x, p['stem0_w'], p['stem0_b'], (1, 1, 1), [(0, 0), (2, 2), (2, 2)])
    x = jax.nn.relu(_bn3d(x, p['stem0_g'], p['stem0_be']))
    x = _maxpool_122(x)
    x = _conv3d(x, p['stem1_w'], p['stem1_b'], (1, 1, 1), [(1, 1), (1, 1), (1, 1)])
    x = jax.nn.relu(_bn3d(x, p['stem1_g'], p['stem1_be']))
    x = _maxpool_122(x)
    x = _conv3d(x, p['stem2_w'], p['stem2_b'], (1, 1, 1), [(1, 1), (1, 1), (1, 1)])
    x = jax.nn.relu(_bn3d(x, p['stem2_g'], p['stem2_be']))
    x = _maxpool_122(x)
    x = _conv3d(x, p['pe_w'], p['pe_b'], (4, 4, 4), [(0, 0), (0, 0), (0, 0)])
    Lt, Lh, Lw = x.shape[2], x.shape[3], x.shape[4]
    N = Lh * Lw
    xp = jnp.transpose(x, (2, 0, 3, 4, 1)).reshape(Lt * b * N, DIM)

    xs = _tf_stack(xp, p['blocks'])

    x_out = jnp.transpose(xs.reshape(Lt, b, N, DIM), (1, 3, 0, 2)).reshape(b, DIM, Lt, Lh, Lw)
    u = jnp.repeat(x_out, 2, axis=2)
    u = _conv3d(u, p['up1_w'], p['up1_b'], (1, 1, 1), [(1, 1), (0, 0), (0, 0)])
    u = jax.nn.elu(_bn3d(u, p['up1_g'], p['up1_be']))
    u = jnp.repeat(u, 2, axis=2)
    u = _conv3d(u, p['up2_w'], p['up2_b'], (1, 1, 1), [(1, 1), (0, 0), (0, 0)])
    u = jax.nn.elu(_bn3d(u, p['up2_g'], p['up2_be']))
    fm = u.mean(axis=3).mean(axis=3)
    rppg = jnp.einsum('oc,bct->bot', p['last_w'][:, :, 0], fm) + p['last_b'][None, :, None]
    return rppg[:, 0, :]


# speedup vs baseline: 1.0239x; 1.0239x over previous
import jax
import jax.numpy as jnp
from jax.experimental import pallas as pl
from jax.experimental.pallas import tpu as pltpu

TAU = 2.0
VTH = 1.0
EPS = 1e-5
DIM = 64
NUM_HEADS = 4
N_WIN = 8
TOPK = 4


def _lif(v):
    return jnp.where(v / TAU - VTH >= 0.0, 1.0, 0.0).astype(v.dtype)


def _mm(a, b):
    return jnp.dot(a.astype(jnp.bfloat16), b.astype(jnp.bfloat16),
                   preferred_element_type=jnp.float32)


def _mmt(a, b):
    return jax.lax.dot_general(a.astype(jnp.bfloat16), b.astype(jnp.bfloat16),
                               (((1,), (1,)), ((), ())),
                               preferred_element_type=jnp.float32)


def _bn_cols(x, g, b):
    m = jnp.mean(x, axis=0, keepdims=True)
    v = jnp.mean(jnp.square(x - m), axis=0, keepdims=True)
    return (x - m) / jnp.sqrt(v + EPS) * g + b


def _elu(x):
    return jnp.where(x > 0.0, x, jnp.exp(x) - 1.0)


def _bn_rows(xe, xo, g, b):
    n = 2.0 * xe.shape[1]
    m = (jnp.sum(xe, axis=1, keepdims=True) + jnp.sum(xo, axis=1, keepdims=True)) / n
    v = (jnp.sum(jnp.square(xe - m), axis=1, keepdims=True)
         + jnp.sum(jnp.square(xo - m), axis=1, keepdims=True)) / n
    s = g / jnp.sqrt(v + EPS)
    return (xe - m) * s + b, (xo - m) * s + b


def _shift_r(x, k):
    z = jnp.zeros((x.shape[0], k), x.dtype)
    return jnp.concatenate([z, x[:, :-k]], axis=1)


def _shift_l(x, k):
    z = jnp.zeros((x.shape[0], k), x.dtype)
    return jnp.concatenate([x[:, k:], z], axis=1)


def _tf_stack_kernel(tb, xin_ref, *args):
    wrefs = args[:30]
    urefs = args[30:44]
    out_ref = args[44]
    qkv_s = args[45]
    att_s = args[46]

    C = DIM
    H = NUM_HEADS
    dh = C // H
    N = 16
    w = N // N_WIN

    x = _lif(xin_ref[...])

    row16 = jax.lax.broadcasted_iota(jnp.int32, (16, 16), 0)
    col16 = jax.lax.broadcasted_iota(jnp.int32, (16, 16), 1)
    row_even = (row16 % 2) == 0

    for blk in range(3):
        g1, b1, Wqkv, Wo, g2, b2, W1, bb1, W2, bb2 = wrefs[blk * 10:(blk + 1) * 10]
        y = _bn_cols(x, g1[...], b1[...])
        qkv_s[...] = _mm(y, Wqkv[...])

        def body(t, carry):
            base = t * N
            qt = qkv_s[pl.ds(base, N), 0:C]
            kt = qkv_s[pl.ds(base, N), C:2 * C]
            vt = qkv_s[pl.ds(base, N), 2 * C:3 * C]
            def pool_rep(m):
                swp = jnp.where(row_even[:, :1],
                                jnp.concatenate([m[1:], m[:1]], axis=0),
                                jnp.concatenate([m[-1:], m[:-1]], axis=0))
                return (m + swp) * 0.5

            qp = pool_rep(qt)
            kp = pool_rep(kt)
            r = _mmt(qp, kp)
            cnt = jnp.zeros((16, 16), jnp.float32)
            for kk in range(8):
                rk = r[:, 2 * kk:2 * kk + 1]
                gt = jnp.where(rk > r, 1.0, 0.0)
                eq = jnp.where((rk == r) & (col16 // 2 > kk), 1.0, 0.0)
                cnt = cnt + gt + eq
            keep = jnp.where(cnt < float(TOPK), 1.0, 0.0)
            amask = (keep - 1.0) * 1e30
            for h in range(H):
                qh = qt[:, h * dh:(h + 1) * dh]
                kh = kt[:, h * dh:(h + 1) * dh]
                vh = vt[:, h * dh:(h + 1) * dh]
                s = _mmt(qh, kh) / 4.0 + amask
                s = s - jnp.max(s, axis=1, keepdims=True)
                e = jnp.exp(s)
                p = e / jnp.sum(e, axis=1, keepdims=True)
                att_s[pl.ds(base, N), h * dh:(h + 1) * dh] = _mm(p, vh)
            return carry

        jax.lax.fori_loop(0, tb, body, 0)
        x = x + _mm(att_s[...], Wo[...])
        z = _bn_cols(x, g2[...], b2[...])
        f = _lif(_mm(z, W1[...]) + bb1[...])
        x = x + _lif(_mm(f, W2[...]) + bb2[...])

    (a1, b1_, c1, ub1, ug1, ube1,
     a2, b2_, c2, ub2, ug2, ube2, lw, lb) = [r[...] for r in urefs]
    xsT = x.T
    dn = xsT.shape[1]

    def upconv(a, bm, cm, bias, xt):
        ye = _mm(a, _shift_r(xt, 16)) + _mm(bm, xt) + _mm(cm, xt) + bias
        yo = _mm(a, xt) + _mm(bm, xt) + _mm(cm, _shift_l(xt, 16)) + bias
        return ye, yo

    y1e, y1o = upconv(a1, b1_, c1, ub1, xsT)
    y1e, y1o = _bn_rows(y1e, y1o, ug1, ube1)
    u1e, u1o = _elu(y1e), _elu(y1o)

    ri = jax.lax.broadcasted_iota(jnp.int32, (dn, 2 * dn), 0)
    ci = jax.lax.broadcasted_iota(jnp.int32, (dn, 2 * dn), 1)
    se = jnp.where(((ci // 16) == 2 * (ri // 16)) & (ci % 16 == ri % 16), 1.0, 0.0)
    so = jnp.where(((ci // 16) == 2 * (ri // 16) + 1) & (ci % 16 == ri % 16), 1.0, 0.0)
    u1 = jnp.dot(u1e, se, preferred_element_type=jnp.float32) \
        + jnp.dot(u1o, so, preferred_element_type=jnp.float32)

    y2e, y2o = upconv(a2, b2_, c2, ub2, u1)
    y2e, y2o = _bn_rows(y2e, y2o, ug2, ube2)
    u2e, u2o = _elu(y2e), _elu(y2o)

    def mean16(u):
        s = u
        for k in (1, 2, 4, 8):
            s = s + _shift_l(s, k)
        return s / 16.0

    rpe = _mm(lw, mean16(u2e)) + lb
    rpo = _mm(lw, mean16(u2o)) + lb

    li = jax.lax.broadcasted_iota(jnp.int32, (2 * dn, 4 * dn // 16), 0)
    ti = jax.lax.broadcasted_iota(jnp.int32, (2 * dn, 4 * dn // 16), 1)
    sel_e = jnp.where((ti % 2 == 0) & (li == 8 * ti), 1.0, 0.0)
    sel_o = jnp.where((ti % 2 == 1) & (li == 8 * (ti - 1)), 1.0, 0.0)
    out_ref[...] = jnp.dot(rpe, sel_e, preferred_element_type=jnp.float32) \
        + jnp.dot(rpo, sel_o, preferred_element_type=jnp.float32)


def _tf_stack(xsnn_pre, p):
    rows = xsnn_pre.shape[0]
    tb = rows // 16
    wargs = []
    for blk in p['blocks']:
        wargs += [blk['bn1_g'].reshape(1, DIM), blk['bn1_b'].reshape(1, DIM),
                  blk['Wqkv'], blk['Wo'],
                  blk['bn2_g'].reshape(1, DIM), blk['bn2_b'].reshape(1, DIM),
                  blk['ffn1_w'], blk['ffn1_b'].reshape(1, 4 * DIM),
                  blk['ffn2_w'], blk['ffn2_b'].reshape(1, DIM)]
    for wname, bias, g, be, co in (('up1', 'up1_b', 'up1_g', 'up1_be', 64),
                                   ('up2', 'up2_b', 'up2_g', 'up2_be', 32)):
        w = p[wname + '_w'][:, :, :, 0, 0]
        wargs += [w[:, :, 0], w[:, :, 1], w[:, :, 2],
                  p[bias].reshape(co, 1), p[g].reshape(co, 1), p[be].reshape(co, 1)]
    wargs += [p['last_w'][:, :, 0], p['last_b'].reshape(1, 1)]
    import functools
    return pl.pallas_call(
        functools.partial(_tf_stack_kernel, tb),
        out_shape=jax.ShapeDtypeStruct((1, 4 * tb), jnp.float32),
        scratch_shapes=[pltpu.VMEM((rows, 3 * DIM), jnp.float32),
                        pltpu.VMEM((rows, DIM), jnp.float32)],
    )(xsnn_pre, *wargs)


def _conv3d(x, w, b, stride, padding):
    y = jax.lax.conv_general_dilated(x, w, window_strides=stride, padding=padding,
                                     dimension_numbers=('NCDHW', 'OIDHW', 'NCDHW'))
    return y + b[None, :, None, None, None]


def _bn3d(x, g, b):
    m = x.mean(axis=(0, 2, 3, 4), keepdims=True)
    v = x.var(axis=(0, 2, 3, 4), keepdims=True)
    return (x - m) / jnp.sqrt(v + EPS) * g[None, :, None, None, None] + b[None, :, None, None, None]


def _maxpool_122(x):
    return jax.lax.reduce_window(x, -jnp.inf, jax.lax.max, (1, 1, 1, 2, 2), (1, 1, 1, 2, 2), 'VALID')


def kernel(x, params):
    p = params
    b = x.shape[0]
    x = _conv3d(x, p['stem0_w'], p['stem0_b'], (1, 1, 1), [(0, 0), (2, 2), (2, 2)])
    x = jax.nn.relu(_bn3d(x, p['stem0_g'], p['stem0_be']))
    x = _maxpool_122(x)
    x = _conv3d(x, p['stem1_w'], p['stem1_b'], (1, 1, 1), [(1, 1), (1, 1), (1, 1)])
    x = jax.nn.relu(_bn3d(x, p['stem1_g'], p['stem1_be']))
    x = _maxpool_122(x)
    x = _conv3d(x, p['stem2_w'], p['stem2_b'], (1, 1, 1), [(1, 1), (1, 1), (1, 1)])
    x = jax.nn.relu(_bn3d(x, p['stem2_g'], p['stem2_be']))
    x = _maxpool_122(x)
    x = _conv3d(x, p['pe_w'], p['pe_b'], (4, 4, 4), [(0, 0), (0, 0), (0, 0)])
    Lt, Lh, Lw = x.shape[2], x.shape[3], x.shape[4]
    N = Lh * Lw
    xp = jnp.transpose(x, (2, 0, 3, 4, 1)).reshape(Lt * b * N, DIM)
    return _tf_stack(xp, p)
